# baseline (speedup 1.0000x reference)
import functools

import jax
import jax.numpy as jnp
from jax import lax
from jax.experimental import pallas as pl
from jax.experimental.pallas import tpu as pltpu
from jax.experimental.pallas import tpu_sc as plsc

NUM_EMBEDDINGS = 1000000
EMB_DIM = 64
NUM_EDGE_TYPES = 8
LANES = 16

NTILES = (NUM_EMBEDDINGS + 127) // 128
JBT = 32
NJB = (NTILES + JBT - 1) // JBT
LAST_T = NTILES - (NJB - 1) * JBT
NGRP = EMB_DIM // 8
NWIN = NGRP * NJB
RAW_ROWS = NGRP * NTILES * 8
GSTRIDE = NTILES * 8 * 128

CEDGES = 128
EPC = CEDGES * EMB_DIM


def _shuffle(x, idx):
    dnums = lax.GatherDimensionNumbers(
        offset_dims=(), collapsed_slice_dims=(0,), start_index_map=(0,))
    return lax.gather(
        x, idx[:, None], dnums, slice_sizes=(1,),
        mode=lax.GatherScatterMode.PROMISE_IN_BOUNDS)


@functools.lru_cache(maxsize=None)
def _build_pack():
    info = plsc.get_sparse_core_info()
    nc, ns = info.num_cores, info.num_subcores
    nw = nc * ns
    slots = (NWIN + nw - 1) // nw

    mesh = plsc.VectorSubcoreMesh(core_axis_name="c", subcore_axis_name="s")

    @functools.partial(
        pl.kernel,
        mesh=mesh,
        out_type=jax.ShapeDtypeStruct((RAW_ROWS, 128), jnp.float32),
        scratch_types=[
            pltpu.VMEM((8, JBT * 128), jnp.float32),
            pltpu.VMEM((8, JBT * 128), jnp.float32),
            pltpu.VMEM((8, JBT * 128), jnp.float32),
            pltpu.SemaphoreType.DMA,
            pltpu.SemaphoreType.DMA,
            pltpu.SemaphoreType.DMA,
            pltpu.SemaphoreType.DMA,
            pltpu.SemaphoreType.DMA,
            pltpu.SemaphoreType.DMA,
        ],
        compiler_params=pltpu.CompilerParams(disable_bounds_checks=True),
    )
    def pack_kernel(tt_hbm, raw_hbm, st_a, st_b, st_c,
                    sin_a, sin_b, sin_c, sout_a, sout_b, sout_c):
        wid = lax.axis_index("s") * nc + lax.axis_index("c")
        stages = (st_a, st_b, st_c)
        sins = (sin_a, sin_b, sin_c)
        souts = (sout_a, sout_b, sout_c)

        def in_desc(g, jb, ntiles, p):
            width = ntiles * 128
            return pltpu.make_async_copy(
                tt_hbm.at[pl.ds(g * 8, 8), pl.ds(jb * (JBT * 128), width)],
                stages[p].at[:, pl.ds(0, width)], sins[p])

        def out_desc(g, jb, w, p):
            q0 = g * (NTILES * 8) + jb * (JBT * 8)
            return pltpu.make_async_copy(
                stages[p].at[:, pl.ds(w * 128, 128)],
                raw_hbm.at[pl.ds(q0 + w * 8, 8), :], souts[p])

        def guarded(slot, fn):
            win = wid + nw * slot
            g = win // NJB
            jb = win - g * NJB
            valid = jnp.logical_and(win >= 0, win < NWIN)

            @pl.when(jnp.logical_and(valid, jb < NJB - 1))
            def _():
                fn(g, jb, JBT)

            @pl.when(jnp.logical_and(valid, jb == NJB - 1))
            def _():
                fn(g, jb, LAST_T)

        def fire_in(slot, p):
            guarded(slot, lambda g, jb, nt: in_desc(g, jb, nt, p).start())

        def wait_in(slot, p):
            guarded(slot, lambda g, jb, nt: in_desc(g, jb, nt, p).wait())

        def fire_outs(slot, p):
            def go(g, jb, nt):
                for w in range(nt):
                    out_desc(g, jb, w, p).start()
            guarded(slot, go)

        def wait_outs(slot, p):
            def go(g, jb, nt):
                for w in range(nt):
                    out_desc(g, jb, w, p).wait()
            guarded(slot, go)

        fire_in(0, 0)

        def step(k, p):
            pn = (p + 1) % 3
            wait_in(k, p)
            fire_outs(k, p)
            wait_outs(k - 2, pn)
            fire_in(k + 1, pn)

        def tri_body(m, carry):
            for q in range(3):
                step(3 * m + q, q)
            return carry

        n_tri = (slots + 2) // 3
        lax.fori_loop(0, n_tri, tri_body, 0)
        wait_outs(slots - 1, (slots - 1) % 3)

    return pack_kernel


@functools.lru_cache(maxsize=None)
def _build_score(batch: int):
    info = plsc.get_sparse_core_info()
    nc, ns = info.num_cores, info.num_subcores
    nw = nc * ns
    e_per_w = batch // nw
    n_chunks = e_per_w // CEDGES
    blocks_per_chunk = CEDGES // LANES

    mesh = plsc.VectorSubcoreMesh(core_axis_name="c", subcore_axis_name="s")

    @functools.partial(
        pl.kernel,
        mesh=mesh,
        out_type=jax.ShapeDtypeStruct((batch,), jnp.float32),
        scratch_types=[
            pltpu.VMEM((e_per_w,), jnp.int32),
            pltpu.VMEM((e_per_w,), jnp.int32),
            pltpu.VMEM((e_per_w,), jnp.int32),
            pltpu.VMEM((e_per_w * EMB_DIM,), jnp.int32),
            pltpu.VMEM((e_per_w * EMB_DIM,), jnp.int32),
            pltpu.VMEM((EPC,), jnp.float32),
            pltpu.VMEM((EPC,), jnp.float32),
            pltpu.VMEM((EPC,), jnp.float32),
            pltpu.VMEM((EPC,), jnp.float32),
            pltpu.VMEM((EMB_DIM, LANES), jnp.float32),
            pltpu.VMEM((EMB_DIM, LANES), jnp.float32),
            pltpu.VMEM((e_per_w,), jnp.float32),
            pltpu.SemaphoreType.DMA,
            pltpu.SemaphoreType.DMA,
        ],
    )
    def score_kernel(sid_hbm, did_hbm, et_hbm, flat_hbm, tp_hbm, dp_hbm,
                     out_hbm, sid_v, did_v, et_v, eis_v, eid_v,
                     sf_a, sf_b, df_a, df_b, tp_v, dp_v, scores_v,
                     sem_a, sem_b):
        wid = lax.axis_index("s") * nc + lax.axis_index("c")
        ebase = wid * e_per_w

        pltpu.sync_copy(sid_hbm.at[pl.ds(ebase, e_per_w)], sid_v)
        pltpu.sync_copy(did_hbm.at[pl.ds(ebase, e_per_w)], did_v)
        pltpu.sync_copy(et_hbm.at[pl.ds(ebase, e_per_w)], et_v)
        pltpu.sync_copy(tp_hbm, tp_v)
        pltpu.sync_copy(dp_hbm, dp_v)

        cconst = [(c // 8) * GSTRIDE + (c % 8) * 128 for c in range(EMB_DIM)]

        def build_body(b, carry):
            chunk = b // blocks_per_chunk
            off0 = chunk * EPC + (b - chunk * blocks_per_chunk) * LANES
            for ids, ei in ((sid_v, eis_v), (did_v, eid_v)):
                ivec = ids[pl.ds(b * LANES, LANES)]
                base = (lax.shift_right_logical(ivec, 7) * 1024
                        + (ivec & 127))
                for c in range(EMB_DIM):
                    ei[pl.ds(off0 + c * CEDGES, LANES)] = base + cconst[c]
            return carry
        lax.fori_loop(0, e_per_w // LANES, build_body, 0)

        def fire_chunk(chunk, ei, dstbuf, sem):
            pltpu.async_copy(
                flat_hbm.at[ei.at[pl.ds(chunk * EPC, EPC)]],
                dstbuf,
                sem)

        def wait_chunk(chunk, ei, dstbuf, sem):
            pltpu.make_async_copy(
                flat_hbm.at[ei.at[pl.ds(chunk * EPC, EPC)]],
                dstbuf,
                sem).wait()

        def compute_chunk(chunk, sbuf, dbuf):
            def cb(b, carry):
                e0 = chunk * CEDGES + b * LANES
                et_vec = et_v[pl.ds(e0, LANES)]
                acc = jnp.zeros((LANES,), jnp.float32)
                for c in range(EMB_DIM):
                    sl = pl.ds(c * CEDGES + b * LANES, LANES)
                    sv = sbuf[sl]
                    dv = dbuf[sl]
                    tt = _shuffle(tp_v[c], et_vec)
                    dd = _shuffle(dp_v[c], et_vec)
                    acc = acc + (sv + tt) * (dv * dd)
                scores_v[pl.ds(e0, LANES)] = acc
                return carry
            lax.fori_loop(0, blocks_per_chunk, cb, 0)

        sbufs = (sf_a, sf_b)
        dbufs = (df_a, df_b)
        fire_chunk(0, eis_v, sbufs[0], sem_a)
        fire_chunk(0, eid_v, dbufs[0], sem_b)
        for chunk in range(n_chunks):
            if chunk + 1 < n_chunks:
                fire_chunk(chunk + 1, eis_v, sbufs[(chunk + 1) % 2], sem_a)
                fire_chunk(chunk + 1, eid_v, dbufs[(chunk + 1) % 2], sem_b)
            wait_chunk(chunk, eis_v, sbufs[chunk % 2], sem_a)
            wait_chunk(chunk, eid_v, dbufs[chunk % 2], sem_b)
            compute_chunk(chunk, sbufs[chunk % 2], dbufs[chunk % 2])

        pltpu.sync_copy(scores_v, out_hbm.at[pl.ds(ebase, e_per_w)])

    return score_kernel


def kernel(src_dst_pairs, condensed_edge_types, table, src_translation, dst_diag):
    batch = condensed_edge_types.shape[0]
    pairs2 = jnp.asarray(src_dst_pairs, jnp.int32).reshape(batch, 2)
    src_ids = pairs2[:, 0]
    dst_ids = pairs2[:, 1]
    tpad = jnp.pad(src_translation.T, ((0, 0), (0, LANES - NUM_EDGE_TYPES)))
    dpad = jnp.pad(dst_diag.T, ((0, 0), (0, LANES - NUM_EDGE_TYPES)))
    raw = _build_pack()(table.T)
    flat = raw.reshape(-1)
    return _build_score(batch)(
        src_ids,
        dst_ids,
        jnp.asarray(condensed_edge_types, jnp.int32),
        flat,
        tpad,
        dpad,
    )
